# baseline (speedup 1.0000x reference)
import functools

import jax
import jax.numpy as jnp
from jax import lax
from jax.experimental import pallas as pl
from jax.experimental.pallas import tpu as pltpu
from jax.experimental.pallas import tpu_sc as plsc

S, D, M, E = 2048, 768, 3072, 8
EPS = 1e-6
TT = 128
NTMAX = 40
CAP = NTMAX * TT
NW = 32
TPW = S // NW


def _prologue_body(x_ref, nw_ref, gw_ref, gb_ref,
                   xn_ref, probs_ref, dest_ref, te_ref, tv_ref, bl_ref):
    x = x_ref[...]
    ms = jnp.mean(x * x, axis=1, keepdims=True)
    xn = x * lax.rsqrt(ms + EPS) * nw_ref[...]
    xn_ref[...] = xn

    logits = lax.dot_general(xn, gw_ref[...], (((1,), (1,)), ((), ())),
                             preferred_element_type=jnp.float32)
    logits = logits + gb_ref[...]
    mx = jnp.max(logits, axis=1, keepdims=True)
    ex = jnp.exp(logits - mx)
    p = ex / jnp.sum(ex, axis=1, keepdims=True)

    iota_e = lax.broadcasted_iota(jnp.int32, (S, E), 1)
    p1 = jnp.max(p, axis=1, keepdims=True)
    i1 = jnp.min(jnp.where(p == p1, iota_e, E), axis=1, keepdims=True)
    mask1 = iota_e == i1
    pm = jnp.where(mask1, -jnp.inf, p)
    p2 = jnp.max(pm, axis=1, keepdims=True)
    i2 = jnp.min(jnp.where(pm == p2, iota_e, E), axis=1, keepdims=True)
    mask2 = iota_e == i2
    oh1 = mask1.astype(jnp.float32)
    oh2 = mask2.astype(jnp.float32)
    probs_ref[...] = jnp.concatenate([p1, p2], axis=1)

    importance = jnp.sum(p, axis=0, keepdims=True)
    load = jnp.sum(oh1 * p1 + oh2 * p2, axis=0, keepdims=True)

    def _std(v):
        u = v / jnp.sum(v)
        mu = jnp.mean(u)
        return jnp.sqrt(jnp.sum((u - mu) ** 2) / (E - 1))

    bl_ref[...] = jnp.reshape(0.5 * (_std(importance) + _std(load)), (1, 1))

    C = 256
    r = lax.broadcasted_iota(jnp.int32, (C, C), 0)
    c = lax.broadcasted_iota(jnp.int32, (C, C), 1)
    tri = (r > c).astype(jnp.float32)

    def _ranks(oh):
        carry = jnp.zeros((1, E), jnp.float32)
        parts = []
        for ci in range(S // C):
            blk = lax.slice(oh, (ci * C, 0), ((ci + 1) * C, E))
            excl = lax.dot_general(tri, blk, (((1,), (0,)), ((), ())),
                                   preferred_element_type=jnp.float32) + carry
            parts.append(jnp.sum(excl * blk, axis=1, keepdims=True))
            carry = carry + jnp.sum(blk, axis=0, keepdims=True)
        return jnp.concatenate(parts, axis=0), carry

    rank1, total1 = _ranks(oh1)
    rank2, total2 = _ranks(oh2)
    count = total1 + total2

    tiles = jnp.floor((count + (TT - 1)) / TT)
    er = lax.broadcasted_iota(jnp.int32, (E, E), 0)
    ec = lax.broadcasted_iota(jnp.int32, (E, E), 1)
    triE = (er < ec).astype(jnp.float32)
    cum_excl = lax.dot_general(tiles, triE, (((1,), (0,)), ((), ())),
                               preferred_element_type=jnp.float32)
    pad_off = TT * cum_excl

    po1 = jnp.sum(oh1 * pad_off, axis=1, keepdims=True)
    po2 = jnp.sum(oh2 * pad_off, axis=1, keepdims=True)
    t1_at_e2 = jnp.sum(oh2 * total1, axis=1, keepdims=True)
    dest1 = (po1 + rank1).astype(jnp.int32)
    dest2 = (po2 + t1_at_e2 + rank2).astype(jnp.int32)
    dest_ref[...] = jnp.concatenate([dest1, dest2], axis=1)

    total_tiles = jnp.sum(tiles)
    ends = cum_excl + tiles
    jt = lax.broadcasted_iota(jnp.int32, (NTMAX, 1), 0).astype(jnp.float32)
    jE = lax.broadcasted_iota(jnp.int32, (NTMAX, E), 0).astype(jnp.float32)
    te_raw = jnp.sum((jE >= ends).astype(jnp.float32), axis=1, keepdims=True)
    te_last = jnp.sum((ends <= total_tiles - 1).astype(jnp.float32))
    valid = jt < total_tiles
    te_ref[...] = jnp.where(valid, te_raw, te_last).astype(jnp.int32)
    tv_ref[...] = valid.astype(jnp.int32)


def _prologue(x2d, norm_w, gate_W, gate_b, interpret=False):
    out_shapes = (
        jax.ShapeDtypeStruct((S, D), jnp.float32),
        jax.ShapeDtypeStruct((S, 2), jnp.float32),
        jax.ShapeDtypeStruct((S, 2), jnp.int32),
        jax.ShapeDtypeStruct((NTMAX, 1), jnp.int32),
        jax.ShapeDtypeStruct((NTMAX, 1), jnp.int32),
        jax.ShapeDtypeStruct((1, 1), jnp.float32),
    )
    return pl.pallas_call(
        _prologue_body,
        out_shape=out_shapes,
        interpret=interpret,
    )(x2d, norm_w.reshape(1, D), gate_W, gate_b.reshape(1, E))


def _mlp_body(te_ref, tv_ref, xg_ref, w1_ref, b1_ref, w2_ref, b2_ref, out_ref):
    t = pl.program_id(0)
    valid = tv_ref[t, 0] == 1

    @pl.when(valid)
    def _():
        xb = xg_ref[...]
        h = lax.dot_general(xb, w1_ref[0], (((1,), (1,)), ((), ())),
                            preferred_element_type=jnp.float32)
        h = h + b1_ref[0]
        h = 0.5 * h * (1.0 + lax.erf(h * 0.7071067811865476))
        y = lax.dot_general(h, w2_ref[0], (((1,), (1,)), ((), ())),
                            preferred_element_type=jnp.float32)
        out_ref[...] = y + b2_ref[0]


def _mlp(te, tv, gathered, W1, b1, W2, b2, interpret=False):
    grid_spec = pltpu.PrefetchScalarGridSpec(
        num_scalar_prefetch=2,
        grid=(NTMAX,),
        in_specs=[
            pl.BlockSpec((TT, D), lambda t, te, tv: (t, 0)),
            pl.BlockSpec((1, M, D), lambda t, te, tv: (te[t, 0], 0, 0)),
            pl.BlockSpec((1, 1, M), lambda t, te, tv: (te[t, 0], 0, 0)),
            pl.BlockSpec((1, D, M), lambda t, te, tv: (te[t, 0], 0, 0)),
            pl.BlockSpec((1, 1, D), lambda t, te, tv: (te[t, 0], 0, 0)),
        ],
        out_specs=pl.BlockSpec((TT, D), lambda t, te, tv: (t, 0)),
    )
    return pl.pallas_call(
        _mlp_body,
        grid_spec=grid_spec,
        out_shape=jax.ShapeDtypeStruct((CAP, D), jnp.float32),
        compiler_params=pltpu.CompilerParams(
            dimension_semantics=("arbitrary",)),
        interpret=interpret,
    )(te, tv, gathered, W1, b1.reshape(E, 1, M), W2, b2.reshape(E, 1, D))


def _sc_dispatch(x_norm, dest_sc):
    mesh = plsc.VectorSubcoreMesh(core_axis_name="c", subcore_axis_name="s")

    @functools.partial(
        pl.kernel,
        out_type=jax.ShapeDtypeStruct((CAP, D), jnp.float32),
        mesh=mesh,
        scratch_types=[
            pltpu.VMEM((TPW,), jnp.int32),
            pltpu.VMEM((TPW,), jnp.int32),
            pltpu.VMEM((TPW, D), jnp.float32),
            pltpu.SemaphoreType.DMA,
            pltpu.SemaphoreType.DMA,
        ],
    )
    def dispatch(xn_hbm, dest_hbm, gat_hbm, idx1_v, idx2_v, rows_v, sem1, sem2):
        w = lax.axis_index("s") * 2 + lax.axis_index("c")
        base = w * TPW
        pltpu.sync_copy(xn_hbm.at[pl.ds(base, TPW)], rows_v)
        pltpu.sync_copy(dest_hbm.at[w, 0], idx1_v)
        pltpu.sync_copy(dest_hbm.at[w, 1], idx2_v)
        c1 = pltpu.async_copy(rows_v, gat_hbm.at[idx1_v], sem1)
        c2 = pltpu.async_copy(rows_v, gat_hbm.at[idx2_v], sem2)
        c1.wait()
        c2.wait()

    return dispatch(x_norm, dest_sc)


def _sc_combine(y, dest_sc):
    mesh = plsc.VectorSubcoreMesh(core_axis_name="c", subcore_axis_name="s")

    @functools.partial(
        pl.kernel,
        out_type=(jax.ShapeDtypeStruct((S, D), jnp.float32),
                  jax.ShapeDtypeStruct((S, D), jnp.float32)),
        mesh=mesh,
        scratch_types=[
            pltpu.VMEM((TPW,), jnp.int32),
            pltpu.VMEM((TPW,), jnp.int32),
            pltpu.VMEM((TPW, D), jnp.float32),
            pltpu.VMEM((TPW, D), jnp.float32),
            pltpu.SemaphoreType.DMA,
            pltpu.SemaphoreType.DMA,
        ],
    )
    def combine(y_hbm, dest_hbm, y0_hbm, y1_hbm,
                idx1_v, idx2_v, buf1_v, buf2_v, sem1, sem2):
        w = lax.axis_index("s") * 2 + lax.axis_index("c")
        base = w * TPW
        pltpu.sync_copy(dest_hbm.at[w, 0], idx1_v)
        pltpu.sync_copy(dest_hbm.at[w, 1], idx2_v)
        c1 = pltpu.async_copy(y_hbm.at[idx1_v], buf1_v, sem1)
        c2 = pltpu.async_copy(y_hbm.at[idx2_v], buf2_v, sem2)
        c1.wait()
        c2.wait()
        pltpu.sync_copy(buf1_v, y0_hbm.at[pl.ds(base, TPW)])
        pltpu.sync_copy(buf2_v, y1_hbm.at[pl.ds(base, TPW)])

    return combine(y, dest_sc)


def _epilogue_body(y0_ref, y1_ref, p_ref, out_ref):
    p = p_ref[...]
    out_ref[...] = y0_ref[...] * p[:, 0:1] + y1_ref[...] * p[:, 1:2]


def _epilogue(y0, y1, probs, interpret=False):
    nt = 4
    blk = S // nt
    return pl.pallas_call(
        _epilogue_body,
        grid=(nt,),
        in_specs=[
            pl.BlockSpec((blk, D), lambda t: (t, 0)),
            pl.BlockSpec((blk, D), lambda t: (t, 0)),
            pl.BlockSpec((blk, 2), lambda t: (t, 0)),
        ],
        out_specs=pl.BlockSpec((blk, D), lambda t: (t, 0)),
        out_shape=jax.ShapeDtypeStruct((S, D), jnp.float32),
        interpret=interpret,
    )(y0, y1, probs)


def kernel(x, norm_w, gate_W, gate_b, W1, b1, W2, b2):
    x2d = x.reshape(S, D)
    x_norm, probs, dest, te, tv, bl = _prologue(x2d, norm_w, gate_W, gate_b)

    dest_sc = dest.T.reshape(2, NW, TPW).transpose(1, 0, 2)

    gathered = _sc_dispatch(x_norm, dest_sc)
    y = _mlp(te, tv, gathered, W1, b1, W2, b2)
    y0, y1 = _sc_combine(y, dest_sc)
    out = _epilogue(y0, y1, probs)
    return out.reshape(1, S, D), bl.reshape(())


# speedup vs baseline: 1.4410x; 1.4410x over previous
import functools

import jax
import jax.numpy as jnp
from jax import lax
from jax.experimental import pallas as pl
from jax.experimental.pallas import tpu as pltpu
from jax.experimental.pallas import tpu_sc as plsc

S, D, M, E = 2048, 768, 3072, 8
EPS = 1e-6
TT = 256
NTMAX = 24
CAP = NTMAX * TT
NW = 32
TPW = S // NW


def _prologue_body(x_ref, nw_ref, gw_ref, gb_ref,
                   xn_ref, probs_ref, dest_ref, te_ref, tv_ref, bl_ref):
    x = x_ref[...]
    ms = jnp.mean(x * x, axis=1, keepdims=True)
    xn = x * lax.rsqrt(ms + EPS) * nw_ref[...]
    xn_ref[...] = xn

    logits = lax.dot_general(xn, gw_ref[...], (((1,), (1,)), ((), ())),
                             preferred_element_type=jnp.float32)
    logits = logits + gb_ref[...]
    mx = jnp.max(logits, axis=1, keepdims=True)
    ex = jnp.exp(logits - mx)
    p = ex / jnp.sum(ex, axis=1, keepdims=True)

    iota_e = lax.broadcasted_iota(jnp.int32, (S, E), 1)
    p1 = jnp.max(p, axis=1, keepdims=True)
    i1 = jnp.min(jnp.where(p == p1, iota_e, E), axis=1, keepdims=True)
    mask1 = iota_e == i1
    pm = jnp.where(mask1, -jnp.inf, p)
    p2 = jnp.max(pm, axis=1, keepdims=True)
    i2 = jnp.min(jnp.where(pm == p2, iota_e, E), axis=1, keepdims=True)
    mask2 = iota_e == i2
    oh1 = mask1.astype(jnp.float32)
    oh2 = mask2.astype(jnp.float32)
    probs_ref[...] = jnp.stack([p1 * jnp.ones((S, 128), jnp.float32),
                                p2 * jnp.ones((S, 128), jnp.float32)])

    importance = jnp.sum(p, axis=0, keepdims=True)
    load = jnp.sum(oh1 * p1 + oh2 * p2, axis=0, keepdims=True)

    def _std(v):
        u = v / jnp.sum(v)
        mu = jnp.mean(u)
        return jnp.sqrt(jnp.sum((u - mu) ** 2) / (E - 1))

    bl_ref[...] = jnp.reshape(0.5 * (_std(importance) + _std(load)), (1, 1))

    C = 256
    r = lax.broadcasted_iota(jnp.int32, (C, C), 0)
    c = lax.broadcasted_iota(jnp.int32, (C, C), 1)
    tri = (r > c).astype(jnp.float32)

    def _ranks(oh):
        carry = jnp.zeros((1, E), jnp.float32)
        parts = []
        for ci in range(S // C):
            blk = lax.slice(oh, (ci * C, 0), ((ci + 1) * C, E))
            excl = lax.dot_general(tri, blk, (((1,), (0,)), ((), ())),
                                   preferred_element_type=jnp.float32) + carry
            parts.append(jnp.sum(excl * blk, axis=1, keepdims=True))
            carry = carry + jnp.sum(blk, axis=0, keepdims=True)
        return jnp.concatenate(parts, axis=0), carry

    rank1, total1 = _ranks(oh1)
    rank2, total2 = _ranks(oh2)
    count = total1 + total2

    tiles = jnp.floor((count + (TT - 1)) / TT)
    er = lax.broadcasted_iota(jnp.int32, (E, E), 0)
    ec = lax.broadcasted_iota(jnp.int32, (E, E), 1)
    triE = (er < ec).astype(jnp.float32)
    cum_excl = lax.dot_general(tiles, triE, (((1,), (0,)), ((), ())),
                               preferred_element_type=jnp.float32)
    pad_off = TT * cum_excl

    po1 = jnp.sum(oh1 * pad_off, axis=1, keepdims=True)
    po2 = jnp.sum(oh2 * pad_off, axis=1, keepdims=True)
    t1_at_e2 = jnp.sum(oh2 * total1, axis=1, keepdims=True)
    dest1 = (po1 + rank1).astype(jnp.int32)
    dest2 = (po2 + t1_at_e2 + rank2).astype(jnp.int32)
    dest_ref[...] = jnp.concatenate([dest1, dest2], axis=1)

    total_tiles = jnp.sum(tiles)
    ends = cum_excl + tiles
    jt = lax.broadcasted_iota(jnp.int32, (NTMAX, 1), 0).astype(jnp.float32)
    jE = lax.broadcasted_iota(jnp.int32, (NTMAX, E), 0).astype(jnp.float32)
    te_raw = jnp.sum((jE >= ends).astype(jnp.float32), axis=1, keepdims=True)
    te_last = jnp.sum((ends <= total_tiles - 1).astype(jnp.float32))
    valid = jt < total_tiles
    te_ref[...] = jnp.where(valid, te_raw, te_last).astype(jnp.int32)
    tv_ref[...] = valid.astype(jnp.int32)


def _prologue(x2d, norm_w, gate_W, gate_b, interpret=False):
    out_shapes = (
        jax.ShapeDtypeStruct((S, D), jnp.float32),
        jax.ShapeDtypeStruct((2, S, 128), jnp.float32),
        jax.ShapeDtypeStruct((S, 2), jnp.int32),
        jax.ShapeDtypeStruct((NTMAX, 1), jnp.int32),
        jax.ShapeDtypeStruct((NTMAX, 1), jnp.int32),
        jax.ShapeDtypeStruct((1, 1), jnp.float32),
    )
    return pl.pallas_call(
        _prologue_body,
        out_shape=out_shapes,
        interpret=interpret,
    )(x2d, norm_w.reshape(1, D), gate_W, gate_b.reshape(1, E))


def _mlp_body(te_ref, tv_ref, xg_ref, w1_ref, b1_ref, w2_ref, b2_ref, ps_ref,
              out_ref):
    t = pl.program_id(0)
    valid = tv_ref[t, 0] == 1

    @pl.when(valid)
    def _():
        xb = xg_ref[...]
        h = lax.dot_general(xb, w1_ref[0], (((1,), (1,)), ((), ())),
                            preferred_element_type=jnp.float32)
        h = h + b1_ref[0]
        h = 0.5 * h * (1.0 + lax.erf(h * 0.7071067811865476))
        y = lax.dot_general(h, w2_ref[0], (((1,), (1,)), ((), ())),
                            preferred_element_type=jnp.float32)
        out_ref[...] = (y + b2_ref[0]) * ps_ref[:, 0:1]


def _mlp(te, tv, gathered, W1, b1, W2, b2, pslot, interpret=False):
    grid_spec = pltpu.PrefetchScalarGridSpec(
        num_scalar_prefetch=2,
        grid=(NTMAX,),
        in_specs=[
            pl.BlockSpec((TT, D), lambda t, te, tv: (t, 0)),
            pl.BlockSpec((1, M, D), lambda t, te, tv: (te[t, 0], 0, 0)),
            pl.BlockSpec((1, 1, M), lambda t, te, tv: (te[t, 0], 0, 0)),
            pl.BlockSpec((1, D, M), lambda t, te, tv: (te[t, 0], 0, 0)),
            pl.BlockSpec((1, 1, D), lambda t, te, tv: (te[t, 0], 0, 0)),
            pl.BlockSpec((TT, 128), lambda t, te, tv: (t, 0)),
        ],
        out_specs=pl.BlockSpec((TT, D), lambda t, te, tv: (t, 0)),
    )
    return pl.pallas_call(
        _mlp_body,
        grid_spec=grid_spec,
        out_shape=jax.ShapeDtypeStruct((CAP, D), jnp.float32),
        compiler_params=pltpu.CompilerParams(
            dimension_semantics=("arbitrary",)),
        interpret=interpret,
    )(te, tv, gathered, W1, b1.reshape(E, 1, M), W2, b2.reshape(E, 1, D),
      pslot)


def _sc_dispatch(x_norm, probs, dest_sc):
    mesh = plsc.VectorSubcoreMesh(core_axis_name="c", subcore_axis_name="s")

    @functools.partial(
        pl.kernel,
        out_type=(jax.ShapeDtypeStruct((CAP, D), jnp.float32),
                  jax.ShapeDtypeStruct((CAP, 128), jnp.float32)),
        mesh=mesh,
        scratch_types=[
            pltpu.VMEM((TPW,), jnp.int32),
            pltpu.VMEM((TPW,), jnp.int32),
            pltpu.VMEM((TPW, D), jnp.float32),
            pltpu.VMEM((TPW, 128), jnp.float32),
            pltpu.VMEM((TPW, 128), jnp.float32),
            pltpu.SemaphoreType.DMA,
            pltpu.SemaphoreType.DMA,
            pltpu.SemaphoreType.DMA,
            pltpu.SemaphoreType.DMA,
        ],
    )
    def dispatch(xn_hbm, pp_hbm, dest_hbm, gat_hbm, ps_hbm,
                 idx1_v, idx2_v, rows_v, pv1_v, pv2_v, s1, s2, s3, s4):
        w = lax.axis_index("s") * 2 + lax.axis_index("c")
        base = w * TPW
        pltpu.sync_copy(xn_hbm.at[pl.ds(base, TPW)], rows_v)
        pltpu.sync_copy(pp_hbm.at[0, pl.ds(base, TPW)], pv1_v)
        pltpu.sync_copy(pp_hbm.at[1, pl.ds(base, TPW)], pv2_v)
        pltpu.sync_copy(dest_hbm.at[w, 0], idx1_v)
        pltpu.sync_copy(dest_hbm.at[w, 1], idx2_v)
        c1 = pltpu.async_copy(rows_v, gat_hbm.at[idx1_v], s1)
        c2 = pltpu.async_copy(rows_v, gat_hbm.at[idx2_v], s2)
        c3 = pltpu.async_copy(pv1_v, ps_hbm.at[idx1_v], s3)
        c4 = pltpu.async_copy(pv2_v, ps_hbm.at[idx2_v], s4)
        c1.wait()
        c2.wait()
        c3.wait()
        c4.wait()

    return dispatch(x_norm, probs, dest_sc)


def _sc_combine(y, dest_sc):
    mesh = plsc.VectorSubcoreMesh(core_axis_name="c", subcore_axis_name="s")

    @functools.partial(
        pl.kernel,
        out_type=jax.ShapeDtypeStruct((S, D), jnp.float32),
        mesh=mesh,
        scratch_types=[
            pltpu.VMEM((TPW,), jnp.int32),
            pltpu.VMEM((TPW,), jnp.int32),
            pltpu.VMEM((TPW, D), jnp.float32),
            pltpu.SemaphoreType.DMA,
            pltpu.SemaphoreType.DMA,
        ],
    )
    def combine(y_hbm, dest_hbm, out_hbm, idx1_v, idx2_v, buf_v, sem1, sem2):
        w = lax.axis_index("s") * 2 + lax.axis_index("c")
        base = w * TPW
        pltpu.sync_copy(dest_hbm.at[w, 0], idx1_v)
        pltpu.sync_copy(dest_hbm.at[w, 1], idx2_v)
        pltpu.async_copy(y_hbm.at[idx1_v], buf_v, sem1).wait()
        pltpu.async_copy(y_hbm.at[idx2_v], buf_v, sem2, add=True).wait()
        pltpu.sync_copy(buf_v, out_hbm.at[pl.ds(base, TPW)])

    return combine(y, dest_sc)


def kernel(x, norm_w, gate_W, gate_b, W1, b1, W2, b2):
    x2d = x.reshape(S, D)
    x_norm, probs, dest, te, tv, bl = _prologue(x2d, norm_w, gate_W, gate_b)

    dest_sc = dest.T.reshape(2, NW, TPW).transpose(1, 0, 2)

    gathered, pslot = _sc_dispatch(x_norm, probs, dest_sc)
    y = _mlp(te, tv, gathered, W1, b1, W2, b2, pslot)
    out = _sc_combine(y, dest_sc)
    return out.reshape(1, S, D), bl.reshape(())
